# baseline (speedup 1.0000x reference)
import functools

import jax
import jax.numpy as jnp
from jax import lax
from jax.experimental import pallas as pl
from jax.experimental.pallas import tpu as pltpu
from jax.experimental.pallas import tpu_sc as plsc

NW = 32
CH = 128
J = 5


@functools.partial(jax.jit, static_argnums=(2, 3, 4))
def _emb_lookup(idx3, table, n_ch, D, B):
    mesh = plsc.VectorSubcoreMesh(core_axis_name="c", subcore_axis_name="s")

    @functools.partial(
        pl.kernel,
        mesh=mesh,
        out_type=jax.ShapeDtypeStruct((B, D), jnp.float32),
        scratch_types=[
            pltpu.VMEM((n_ch, CH), jnp.int32),
            pltpu.VMEM((J * CH, D), jnp.float32),
            pltpu.SemaphoreType.DMA,
        ],
        compiler_params=pltpu.CompilerParams(use_tc_tiling_on_sc=False),
    )
    def emb(idx_hbm, table_hbm, out_hbm, idx_v, buf, gsem):
        wid = lax.axis_index("s") * 2 + lax.axis_index("c")
        rows_per_w = n_ch * CH
        base = wid * rows_per_w
        pltpu.sync_copy(idx_hbm.at[wid], idx_v)

        def body(g, carry):
            hs = []
            for j in range(J):
                hs.append(pltpu.async_copy(
                    table_hbm.at[idx_v.at[g * J + j]],
                    buf.at[pl.ds(j * CH, CH)],
                    gsem))
            for h in hs:
                h.wait()
            pltpu.sync_copy(buf, out_hbm.at[pl.ds(base + g * (J * CH), J * CH)])
            return carry

        lax.fori_loop(0, n_ch // J, body, 0)

    return emb(idx3, table)


def kernel(indices, table):
    S0, S1 = indices.shape
    V, D = table.shape
    B = S0 * S1
    assert B % (NW * CH) == 0
    n_ch = B // (NW * CH)
    idx3 = indices.astype(jnp.int32).reshape(NW, n_ch, CH)
    out = _emb_lookup(idx3, table, n_ch, D, B)
    return out.reshape(S0, S1, D)
